# baseline (speedup 1.0000x reference)
import functools

import jax
import jax.numpy as jnp
from jax import lax
from jax.experimental import pallas as pl
from jax.experimental.pallas import tpu as pltpu
from jax.experimental.pallas import tpu_sc as plsc

_D = 1024
_S = 4096
_B = 4
_NC = 2
_SC_ROWS = _S // _NC
_CH = 512
_NP = _SC_ROWS // _CH

_mesh = plsc.ScalarSubcoreMesh(axis_name="c", num_cores=_NC)


@functools.partial(
    pl.kernel,
    mesh=_mesh,
    out_type=jax.ShapeDtypeStruct((_B, _S, _D), jnp.float32),
    scratch_types=[
        pltpu.VMEM_SHARED((_CH, _D), jnp.float32),
    ],
)
def _pos_embed_sc(table_hbm, out_hbm, buf):
    base = lax.axis_index("c") * _SC_ROWS
    for p in range(_NP):
        off = base + p * _CH
        pltpu.sync_copy(table_hbm.at[pl.ds(off, _CH)], buf)
        for b in range(_B):
            pltpu.sync_copy(buf, out_hbm.at[b, pl.ds(off, _CH)])


def kernel(embeddings, table):
    del embeddings
    return _pos_embed_sc(table)


# speedup vs baseline: 1.7615x; 1.7615x over previous
import functools

import jax
import jax.numpy as jnp
from jax import lax
from jax.experimental import pallas as pl
from jax.experimental.pallas import tpu as pltpu
from jax.experimental.pallas import tpu_sc as plsc

_D = 1024
_S = 4096
_B = 4
_NC = 2
_NS = 16
_NW = _NC * _NS
_ROWS_PER_W = _S // _NW
_CH = 64

_mesh = plsc.VectorSubcoreMesh(
    core_axis_name="c", subcore_axis_name="s", num_cores=_NC, num_subcores=_NS
)


@functools.partial(
    pl.kernel,
    mesh=_mesh,
    out_type=jax.ShapeDtypeStruct((_B, _S, _D), jnp.float32),
    scratch_types=[
        pltpu.VMEM((_CH, _D), jnp.float32),
    ],
)
def _pos_embed_sc(table_hbm, out_hbm, buf):
    wid = lax.axis_index("s") * _NC + lax.axis_index("c")
    base = wid * _ROWS_PER_W
    for p in range(_ROWS_PER_W // _CH):
        off = base + p * _CH
        pltpu.sync_copy(table_hbm.at[pl.ds(off, _CH)], buf)
        for b in range(_B):
            pltpu.sync_copy(buf, out_hbm.at[b, pl.ds(off, _CH)])


def kernel(embeddings, table):
    del embeddings
    return _pos_embed_sc(table)
